# baseline (speedup 1.0000x reference)
import jax
import jax.numpy as jnp
from jax.experimental import pallas as pl
from jax.experimental.pallas import tpu as pltpu

_DILATIONS = (1, 2, 4, 8, 16)
_ND = len(_DILATIONS)
_BN_EPS = 1e-3
_HALO = 16
_HB = 16
_W = 256
_WH = _W // 2
_WS = _WH + 2 * _HALO

_INV_SQRT2 = 0.7071067811865476


def _body(xa_ref, xb_ref, xc_ref, kr_ref, sc_ref, sh_ref,
          o0, o1, o2, o3, o4, pt_ref, acc_ref):
    outs = (o0, o1, o2, o3, o4)
    xa = xa_ref[0].reshape(_HB * _WS, 128)
    xb = xb_ref[0].reshape(_HB * _WS, 128)
    xc = xc_ref[0].reshape(_HB * _WS, 128)
    for t in range(9):
        kt = kr_ref[t]
        pa = jnp.dot(xa, kt, preferred_element_type=jnp.float32)
        pb = jnp.dot(xb, kt, preferred_element_type=jnp.float32)
        pc = jnp.dot(xc, kt, preferred_element_type=jnp.float32)
        pt_ref[0:_HB] = pa.reshape(_HB, _WS, 128)
        pt_ref[_HB:2 * _HB] = pb.reshape(_HB, _WS, 128)
        pt_ref[2 * _HB:3 * _HB] = pc.reshape(_HB, _WS, 128)
        ky, kx = divmod(t, 3)
        for di, d in enumerate(_DILATIONS):
            r0 = _HALO + d * (ky - 1)
            c0 = _HALO + d * (kx - 1)
            win = pt_ref[r0:r0 + _HB, c0:c0 + _WH, :]
            if t == 0:
                acc_ref[di] = win
            else:
                acc_ref[di] += win
    for di in range(_ND):
        for r in range(0, _HB, 8):
            y = acc_ref[di, r:r + 8] * sc_ref[di] + sh_ref[di]
            g = 0.5 * y * (1.0 + jax.lax.erf(y * _INV_SQRT2))
            outs[di][0, r:r + 8, 0:_WH, :] = g[:, :, 0:64]
            outs[di][0, r:r + 8, _WH:_W, :] = g[:, :, 64:128]


def kernel(x, kernel, gamma, beta, mov_mean, mov_var):
    B, H, W, C = x.shape
    scale = gamma * jax.lax.rsqrt(mov_var + _BN_EPS)
    shift = beta - mov_mean * scale
    sc2 = jnp.concatenate([scale, scale], axis=-1)
    sh2 = jnp.concatenate([shift, shift], axis=-1)
    xpad = jnp.pad(x, ((0, 0), (_HALO, _HALO), (_HALO, _HALO), (0, 0)))
    xpad = xpad.astype(jnp.bfloat16)
    xp = jnp.concatenate(
        [xpad[:, :, 0:_WS, :], xpad[:, :, _WH:_WH + _WS, :]], axis=-1)
    kb = kernel.reshape(9, C, C).astype(jnp.bfloat16)
    z = jnp.zeros((9, C, C), jnp.bfloat16)
    kr = jnp.concatenate(
        [jnp.concatenate([kb, z], -1), jnp.concatenate([z, kb], -1)], axis=1)

    nh = H // _HB
    grid = (B, nh)
    blk_in = (1, _HB, _WS, 2 * C)
    out_sds = jax.ShapeDtypeStruct((B, H, W, C), jnp.float32)
    out_spec = pl.BlockSpec((1, _HB, W, C), lambda b, i: (b, i, 0, 0))

    outs = pl.pallas_call(
        _body,
        grid=grid,
        in_specs=[
            pl.BlockSpec(blk_in, lambda b, i: (b, i, 0, 0)),
            pl.BlockSpec(blk_in, lambda b, i: (b, i + 1, 0, 0)),
            pl.BlockSpec(blk_in, lambda b, i: (b, i + 2, 0, 0)),
            pl.BlockSpec((9, 2 * C, 2 * C), lambda b, i: (0, 0, 0)),
            pl.BlockSpec((_ND, 2 * C), lambda b, i: (0, 0)),
            pl.BlockSpec((_ND, 2 * C), lambda b, i: (0, 0)),
        ],
        out_specs=[out_spec] * _ND,
        out_shape=[out_sds] * _ND,
        scratch_shapes=[
            pltpu.VMEM((3 * _HB, _WS, 2 * C), jnp.float32),
            pltpu.VMEM((_ND, _HB, _WH, 2 * C), jnp.float32),
        ],
        compiler_params=pltpu.CompilerParams(
            dimension_semantics=("parallel", "arbitrary"),
            vmem_limit_bytes=56 * 1024 * 1024,
        ),
        name="shared_tap_dilated_conv",
    )(xp, xp, xp, kr, sc2, sh2)
    return tuple(outs)
